# Initial kernel scaffold; baseline (speedup 1.0000x reference)
import functools

import jax
import jax.numpy as jnp
from jax import lax
from jax.experimental import pallas as pl
from jax.experimental.pallas import tpu as pltpu
from jax.experimental.pallas import tpu_sc as plsc

_EPS = 1e-5


def _ln(h, g, b):
    mu = jnp.mean(h, axis=-1, keepdims=True)
    var = jnp.mean((h - mu) ** 2, axis=-1, keepdims=True)
    return (h - mu) * lax.rsqrt(var + _EPS) * g + b


def _mlp_conv16(x, W1, b1, ln1_g, ln1_b, W2, b2, ln2_g, ln2_b, ED):
    N, D = x.shape
    H = W1.shape[1]
    BN = 1000
    grid = (N // BN,)

    def body(x_ref, w1_ref, b1_ref, g1_ref, bb1_ref, w2_ref, b2_ref, g2_ref,
             bb2_ref, o_ref):
        xb = x_ref[...]
        h = jnp.dot(xb, w1_ref[...], preferred_element_type=jnp.float32)
        h = _ln(h + b1_ref[...], g1_ref[...], bb1_ref[...])
        h = jnp.maximum(h, 0.0)
        h2 = jnp.dot(h, w2_ref[...], preferred_element_type=jnp.float32)
        h2 = _ln(h2 + b2_ref[...], g2_ref[...], bb2_ref[...])
        o_ref[...] = h2[:, :ED]

    row_spec = lambda r: pl.BlockSpec((1, r), lambda i: (0, 0))
    return pl.pallas_call(
        body,
        grid=grid,
        in_specs=[
            pl.BlockSpec((BN, D), lambda i: (i, 0)),
            pl.BlockSpec((D, H), lambda i: (0, 0)),
            row_spec(H), row_spec(H), row_spec(H),
            pl.BlockSpec((H, D), lambda i: (0, 0)),
            row_spec(D), row_spec(D), row_spec(D),
        ],
        out_specs=pl.BlockSpec((BN, ED), lambda i: (i, 0)),
        out_shape=jax.ShapeDtypeStruct((N, ED), jnp.float32),
    )(x, W1, b1.reshape(1, H), ln1_g.reshape(1, H), ln1_b.reshape(1, H),
      W2, b2.reshape(1, D), ln2_g.reshape(1, D), ln2_b.reshape(1, D))


def _sc_edge_agg(conv16, src2d, dst2d, ea, N):
    ED = conv16.shape[1]
    NC, NS = 2, 16
    NW = NC * NS
    E_pad = ea.shape[0]
    P = E_pad // NW
    K = P // 128
    NG = 16
    NCHUNK = K // NG
    CH = NG * 128
    RPS = N // NS

    mesh = plsc.VectorSubcoreMesh(core_axis_name="c", subcore_axis_name="s")

    @functools.partial(
        pl.kernel,
        out_type=jax.ShapeDtypeStruct((NC, N, ED), jnp.float32),
        mesh=mesh,
        scratch_types=[
            pltpu.VMEM((NG, 128), jnp.int32),
            pltpu.VMEM((NG, 128), jnp.int32),
            pltpu.VMEM((CH, ED), jnp.float32),
            pltpu.VMEM((CH, ED), jnp.float32),
            pltpu.VMEM((RPS, ED), jnp.float32),
            pltpu.VMEM_SHARED((N, ED), jnp.float32),
            pltpu.SemaphoreType.DMA,
        ],
    )
    def sck(conv_hbm, src_hbm, dst_hbm, ea_hbm, out_hbm,
            src_v, dst_v, ea_v, rows_v, stripe_v, agg_sh, sem):
        c = lax.axis_index("c")
        s = lax.axis_index("s")
        wid = c * NS + s

        zero = jnp.zeros((ED,), jnp.float32)

        @plsc.parallel_loop(0, RPS, unroll=8)
        def _(i):
            stripe_v[i] = zero

        pltpu.sync_copy(stripe_v, agg_sh.at[pl.ds(s * RPS, RPS)])
        plsc.subcore_barrier()

        def chunk(t, carry):
            grow = wid * K + t * NG
            ebase = wid * P + t * CH
            pltpu.sync_copy(src_hbm.at[pl.ds(grow, NG)], src_v)
            pltpu.sync_copy(dst_hbm.at[pl.ds(grow, NG)], dst_v)
            pltpu.sync_copy(ea_hbm.at[pl.ds(ebase, CH)], ea_v)
            gathers = []
            for j in range(NG):
                gathers.append(pltpu.async_copy(
                    conv_hbm.at[dst_v.at[j]],
                    rows_v.at[pl.ds(j * 128, 128)], sem))
            for g in gathers:
                g.wait()

            @plsc.parallel_loop(0, CH, unroll=8)
            def _(i):
                rows_v[i] = rows_v[i] * ea_v[i]

            for j in range(NG):
                pltpu.sync_copy(rows_v.at[pl.ds(j * 128, 128)],
                                agg_sh.at[src_v.at[j]], add=True)
            return carry

        lax.fori_loop(0, NCHUNK, chunk, 0)
        plsc.subcore_barrier()

        pltpu.sync_copy(agg_sh.at[pl.ds(s * RPS, RPS)], stripe_v)
        pltpu.sync_copy(stripe_v, out_hbm.at[c].at[pl.ds(s * RPS, RPS)])

    return sck(conv16, src2d, dst2d, ea)


def _final_ln(x, agg2, out_g, out_b):
    N, D = x.shape
    ED = agg2.shape[2]
    BN = 1000
    grid = (N // BN,)

    def body(x_ref, a_ref, g_ref, b_ref, o_ref):
        a = a_ref[0] + a_ref[1]
        h = x_ref[...] + jnp.concatenate(
            [a, jnp.zeros((BN, D - ED), jnp.float32)], axis=1)
        o_ref[...] = _ln(h, g_ref[...], b_ref[...])

    return pl.pallas_call(
        body,
        grid=grid,
        in_specs=[
            pl.BlockSpec((BN, D), lambda i: (i, 0)),
            pl.BlockSpec((2, BN, ED), lambda i: (0, i, 0)),
            pl.BlockSpec((1, D), lambda i: (0, 0)),
            pl.BlockSpec((1, D), lambda i: (0, 0)),
        ],
        out_specs=pl.BlockSpec((BN, D), lambda i: (i, 0)),
        out_shape=jax.ShapeDtypeStruct((N, D), jnp.float32),
    )(x, agg2, out_g.reshape(1, D), out_b.reshape(1, D))


def kernel(x, edge_index, edge_attr, W1, b1, ln1_g, ln1_b, W2, b2, ln2_g,
           ln2_b, out_g, out_b):
    N, D = x.shape
    E, ED = edge_attr.shape
    NW, GROUP = 32, 128

    conv16 = _mlp_conv16(x, W1, b1, ln1_g, ln1_b, W2, b2, ln2_g, ln2_b, ED)

    P = -(-E // (NW * GROUP)) * GROUP
    E_pad = NW * P
    pad = E_pad - E
    src = jnp.pad(edge_index[0], (0, pad)).reshape(E_pad // GROUP, GROUP)
    dst = jnp.pad(edge_index[1], (0, pad)).reshape(E_pad // GROUP, GROUP)
    ea = jnp.pad(edge_attr, ((0, pad), (0, 0)))

    agg2 = _sc_edge_agg(conv16, src, dst, ea, N)
    return _final_ln(x, agg2, out_g, out_b)


# speedup vs baseline: 5.3460x; 5.3460x over previous
import functools

import jax
import jax.numpy as jnp
from jax import lax
from jax.experimental import pallas as pl
from jax.experimental.pallas import tpu as pltpu
from jax.experimental.pallas import tpu_sc as plsc

_EPS = 1e-5
_NG = 16


def _ln(h, g, b):
    mu = jnp.mean(h, axis=-1, keepdims=True)
    var = jnp.mean((h - mu) ** 2, axis=-1, keepdims=True)
    return (h - mu) * lax.rsqrt(var + _EPS) * g + b


def _mlp_conv16(x, W1, b1, ln1_g, ln1_b, W2, b2, ln2_g, ln2_b, ED):
    N, D = x.shape
    H = W1.shape[1]
    BN = 1000
    grid = (N // BN,)

    def body(x_ref, w1_ref, b1_ref, g1_ref, bb1_ref, w2_ref, b2_ref, g2_ref,
             bb2_ref, o_ref):
        xb = x_ref[...]
        h = jnp.dot(xb, w1_ref[...], preferred_element_type=jnp.float32)
        h = _ln(h + b1_ref[...], g1_ref[...], bb1_ref[...])
        h = jnp.maximum(h, 0.0)
        h2 = jnp.dot(h, w2_ref[...], preferred_element_type=jnp.float32)
        h2 = _ln(h2 + b2_ref[...], g2_ref[...], bb2_ref[...])
        o_ref[...] = h2[:, :ED]

    row_spec = lambda r: pl.BlockSpec((1, r), lambda i: (0, 0))
    return pl.pallas_call(
        body,
        grid=grid,
        in_specs=[
            pl.BlockSpec((BN, D), lambda i: (i, 0)),
            pl.BlockSpec((D, H), lambda i: (0, 0)),
            row_spec(H), row_spec(H), row_spec(H),
            pl.BlockSpec((H, D), lambda i: (0, 0)),
            row_spec(D), row_spec(D), row_spec(D),
        ],
        out_specs=pl.BlockSpec((BN, ED), lambda i: (i, 0)),
        out_shape=jax.ShapeDtypeStruct((N, ED), jnp.float32),
    )(x, W1, b1.reshape(1, H), ln1_g.reshape(1, H), ln1_b.reshape(1, H),
      W2, b2.reshape(1, D), ln2_g.reshape(1, D), ln2_b.reshape(1, D))


def _sc_edge_agg(conv16, src2d, dst2d, ea, N_pad):
    ED = conv16.shape[1]
    NC, NS = 2, 16
    NW = NC * NS
    E_pad = ea.shape[0]
    P = E_pad // NW
    K = P // 128
    NG = _NG
    NCHUNK = K // NG
    CH = NG * 128
    RPS = N_pad // NS

    mesh = plsc.VectorSubcoreMesh(core_axis_name="c", subcore_axis_name="s")

    @functools.partial(
        pl.kernel,
        out_type=jax.ShapeDtypeStruct((NC, N_pad, ED), jnp.float32),
        mesh=mesh,
        scratch_types=[
            pltpu.VMEM((NG, 128), jnp.int32),
            pltpu.VMEM((NG, 128), jnp.int32),
            pltpu.VMEM((CH, ED), jnp.float32),
            pltpu.VMEM((CH, ED), jnp.float32),
            pltpu.VMEM((RPS, ED), jnp.float32),
            pltpu.VMEM_SHARED((N_pad, ED), jnp.float32),
            pltpu.SemaphoreType.DMA,
        ],
        compiler_params=pltpu.CompilerParams(use_tc_tiling_on_sc=False),
    )
    def sck(conv_hbm, src_hbm, dst_hbm, ea_hbm, out_hbm,
            src_v, dst_v, ea_v, rows_v, stripe_v, agg_sh, sem):
        c = lax.axis_index("c")
        s = lax.axis_index("s")
        wid = c * NS + s

        zero = jnp.zeros((ED,), jnp.float32)

        @plsc.parallel_loop(0, RPS, unroll=8)
        def _(i):
            stripe_v[i] = zero

        pltpu.sync_copy(stripe_v, agg_sh.at[pl.ds(s * RPS, RPS)])
        plsc.subcore_barrier()

        def chunk(t, carry):
            grow = wid * K + t * NG
            ebase = wid * P + t * CH
            pltpu.sync_copy(src_hbm.at[pl.ds(grow, NG)], src_v)
            pltpu.sync_copy(dst_hbm.at[pl.ds(grow, NG)], dst_v)
            pltpu.sync_copy(ea_hbm.at[pl.ds(ebase, CH)], ea_v)
            gathers = []
            for j in range(NG):
                gathers.append(pltpu.async_copy(
                    conv_hbm.at[dst_v.at[j]],
                    rows_v.at[pl.ds(j * 128, 128)], sem))
            for g in gathers:
                g.wait()

            @plsc.parallel_loop(0, CH, unroll=8)
            def _(i):
                rows_v[i] = rows_v[i] * ea_v[i]

            for j in range(NG):
                pltpu.sync_copy(rows_v.at[pl.ds(j * 128, 128)],
                                agg_sh.at[src_v.at[j]], add=True)
            return carry

        lax.fori_loop(0, NCHUNK, chunk, 0)
        plsc.subcore_barrier()

        pltpu.sync_copy(agg_sh.at[pl.ds(s * RPS, RPS)], stripe_v)
        pltpu.sync_copy(stripe_v, out_hbm.at[c].at[pl.ds(s * RPS, RPS)])

    return sck(conv16, src2d, dst2d, ea)


def _final_ln(x, agg2, out_g, out_b):
    N, D = x.shape
    ED = agg2.shape[2]
    BN = 1000
    grid = (N // BN,)

    def body(x_ref, a_ref, g_ref, b_ref, o_ref):
        a = a_ref[0] + a_ref[1]
        h = x_ref[...] + jnp.concatenate(
            [a, jnp.zeros((BN, D - ED), jnp.float32)], axis=1)
        o_ref[...] = _ln(h, g_ref[...], b_ref[...])

    return pl.pallas_call(
        body,
        grid=grid,
        in_specs=[
            pl.BlockSpec((BN, D), lambda i: (i, 0)),
            pl.BlockSpec((2, BN, ED), lambda i: (0, i, 0)),
            pl.BlockSpec((1, D), lambda i: (0, 0)),
            pl.BlockSpec((1, D), lambda i: (0, 0)),
        ],
        out_specs=pl.BlockSpec((BN, D), lambda i: (i, 0)),
        out_shape=jax.ShapeDtypeStruct((N, D), jnp.float32),
    )(x, agg2, out_g.reshape(1, D), out_b.reshape(1, D))


def kernel(x, edge_index, edge_attr, W1, b1, ln1_g, ln1_b, W2, b2, ln2_g,
           ln2_b, out_g, out_b):
    N, D = x.shape
    E, ED = edge_attr.shape
    NW, GROUP = 32, 128

    conv16 = _mlp_conv16(x, W1, b1, ln1_g, ln1_b, W2, b2, ln2_g, ln2_b, ED)

    quantum = NW * GROUP * _NG
    E_pad = -(-E // quantum) * quantum
    pad = E_pad - E
    src = jnp.pad(edge_index[0], (0, pad)).reshape(E_pad // GROUP, GROUP)
    dst = jnp.pad(edge_index[1], (0, pad)).reshape(E_pad // GROUP, GROUP)
    ea = jnp.pad(edge_attr, ((0, pad), (0, 0)))

    N_pad = -(-N // (16 * 8)) * (16 * 8)
    agg2 = _sc_edge_agg(conv16, src, dst, ea, N_pad)
    return _final_ln(x, agg2, out_g, out_b)


# speedup vs baseline: 9.5597x; 1.7882x over previous
import functools

import jax
import jax.numpy as jnp
from jax import lax
from jax.experimental import pallas as pl
from jax.experimental.pallas import tpu as pltpu
from jax.experimental.pallas import tpu_sc as plsc

_EPS = 1e-5
_NG = 16


def _ln(h, g, b):
    mu = jnp.mean(h, axis=-1, keepdims=True)
    var = jnp.mean((h - mu) ** 2, axis=-1, keepdims=True)
    return (h - mu) * lax.rsqrt(var + _EPS) * g + b


def _mlp_conv16(x, W1, b1, ln1_g, ln1_b, W2, b2, ln2_g, ln2_b, ED):
    N, D = x.shape
    H = W1.shape[1]
    BN = 1000
    grid = (N // BN,)

    def body(x_ref, w1_ref, b1_ref, g1_ref, bb1_ref, w2_ref, b2_ref, g2_ref,
             bb2_ref, o_ref):
        xb = x_ref[...]
        h = jnp.dot(xb, w1_ref[...], preferred_element_type=jnp.float32)
        h = _ln(h + b1_ref[...], g1_ref[...], bb1_ref[...])
        h = jnp.maximum(h, 0.0)
        h2 = jnp.dot(h, w2_ref[...], preferred_element_type=jnp.float32)
        h2 = _ln(h2 + b2_ref[...], g2_ref[...], bb2_ref[...])
        o_ref[...] = h2[:, :ED]

    row_spec = lambda r: pl.BlockSpec((1, r), lambda i: (0, 0))
    return pl.pallas_call(
        body,
        grid=grid,
        in_specs=[
            pl.BlockSpec((BN, D), lambda i: (i, 0)),
            pl.BlockSpec((D, H), lambda i: (0, 0)),
            row_spec(H), row_spec(H), row_spec(H),
            pl.BlockSpec((H, D), lambda i: (0, 0)),
            row_spec(D), row_spec(D), row_spec(D),
        ],
        out_specs=pl.BlockSpec((BN, ED), lambda i: (i, 0)),
        out_shape=jax.ShapeDtypeStruct((N, ED), jnp.float32),
    )(x, W1, b1.reshape(1, H), ln1_g.reshape(1, H), ln1_b.reshape(1, H),
      W2, b2.reshape(1, D), ln2_g.reshape(1, D), ln2_b.reshape(1, D))


def _sc_edge_agg(conv16, ei3, ea, N_pad):
    ED = conv16.shape[1]
    NC, NS = 2, 16
    NW = NC * NS
    G = ei3.shape[1]
    CH = 128
    NB = 4
    MAXG = -(-G // NW)
    MAXG = -(-MAXG // NB) * NB
    NT = MAXG // NB
    RPS = N_pad // NS

    mesh = plsc.VectorSubcoreMesh(core_axis_name="c", subcore_axis_name="s")

    @functools.partial(
        pl.kernel,
        out_type=jax.ShapeDtypeStruct((NC, N_pad, ED), jnp.float32),
        mesh=mesh,
        scratch_types=[
            pltpu.VMEM((NB, CH), jnp.int32),
            pltpu.VMEM((NB, CH), jnp.int32),
            pltpu.VMEM((NB * CH, ED), jnp.float32),
            pltpu.VMEM((NB * CH, ED), jnp.float32),
            pltpu.VMEM((RPS, ED), jnp.float32),
            pltpu.VMEM_SHARED((N_pad, ED), jnp.float32),
            pltpu.SemaphoreType.DMA((2,)),
            pltpu.SemaphoreType.DMA((2,)),
            pltpu.SemaphoreType.DMA((2,)),
        ],
        compiler_params=pltpu.CompilerParams(use_tc_tiling_on_sc=False),
    )
    def sck(conv_hbm, ei_hbm, ea_hbm, out_hbm,
            src_v, dst_v, ea_v, rows_v, stripe_v, agg_sh,
            sem_lin, sem_g, sem_s):
        c = lax.axis_index("c")
        s = lax.axis_index("s")
        wid = c * NS + s

        zero = jnp.zeros((ED,), jnp.float32)

        @plsc.parallel_loop(0, RPS, unroll=8)
        def _(i):
            stripe_v[i] = zero

        pltpu.sync_copy(stripe_v, agg_sh.at[pl.ds(s * RPS, RPS)])
        plsc.subcore_barrier()

        def lin_descs(g, slot):
            gc = jnp.minimum(wid * MAXG + g, G - 1)
            p = slot % 2
            return [
                pltpu.make_async_copy(ei_hbm.at[0, gc], src_v.at[slot],
                                      sem_lin.at[p]),
                pltpu.make_async_copy(ei_hbm.at[1, gc], dst_v.at[slot],
                                      sem_lin.at[p]),
                pltpu.make_async_copy(ea_hbm.at[pl.ds(gc * CH, CH)],
                                      ea_v.at[pl.ds(slot * CH, CH)],
                                      sem_lin.at[p]),
            ]

        def gather_desc(slot):
            return pltpu.make_async_copy(
                conv_hbm.at[dst_v.at[slot]],
                rows_v.at[pl.ds(slot * CH, CH)], sem_g.at[slot % 2])

        def scatter_desc(slot):
            return pltpu.make_async_copy(
                rows_v.at[pl.ds(slot * CH, CH)],
                agg_sh.at[src_v.at[slot]], sem_s.at[slot % 2])

        def multiply(g, slot):
            scale = jnp.where(wid * MAXG + g < G, 1.0, 0.0).astype(jnp.float32)

            @plsc.parallel_loop(0, CH, unroll=8)
            def _(i):
                k = slot * CH + i
                rows_v[k] = rows_v[k] * ea_v[k] * scale

        def step(g, b, scatter_wait, lin_issue, gather_issue):
            if scatter_wait:
                scatter_desc((b + 2) % NB).wait()
            if lin_issue:
                for d in lin_descs(g + 2, (b + 2) % NB):
                    d.start()
            if gather_issue:
                for d in lin_descs(g + 1, (b + 1) % NB):
                    d.wait()
                gather_desc((b + 1) % NB).start()
            gather_desc(b).wait()
            multiply(g, b)
            scatter_desc(b).start(add=True)

        for d in lin_descs(0, 0):
            d.start()
        for d in lin_descs(1, 1):
            d.start()
        for d in lin_descs(0, 0):
            d.wait()
        gather_desc(0).start()

        for b in range(NB):
            step(b, b, scatter_wait=(b >= 2), lin_issue=True,
                 gather_issue=True)

        def super_iter(t, carry):
            for b in range(NB):
                step(t * NB + b, b, scatter_wait=True, lin_issue=True,
                     gather_issue=True)
            return carry

        lax.fori_loop(1, NT - 1, super_iter, 0)

        for b in range(NB):
            g = MAXG - NB + b
            step(g, b, scatter_wait=True, lin_issue=(b < 2),
                 gather_issue=(b < NB - 1))
        scatter_desc(NB - 2).wait()
        scatter_desc(NB - 1).wait()

        plsc.subcore_barrier()

        pltpu.sync_copy(agg_sh.at[pl.ds(s * RPS, RPS)], stripe_v)
        pltpu.sync_copy(stripe_v, out_hbm.at[c].at[pl.ds(s * RPS, RPS)])

    return sck(conv16, ei3, ea)


def _final_ln(x, agg2, out_g, out_b):
    N, D = x.shape
    ED = agg2.shape[2]
    BN = 1000
    grid = (N // BN,)

    def body(x_ref, a_ref, g_ref, b_ref, o_ref):
        a = a_ref[0] + a_ref[1]
        h = x_ref[...] + jnp.concatenate(
            [a, jnp.zeros((BN, D - ED), jnp.float32)], axis=1)
        o_ref[...] = _ln(h, g_ref[...], b_ref[...])

    return pl.pallas_call(
        body,
        grid=grid,
        in_specs=[
            pl.BlockSpec((BN, D), lambda i: (i, 0)),
            pl.BlockSpec((2, BN, ED), lambda i: (0, i, 0)),
            pl.BlockSpec((1, D), lambda i: (0, 0)),
            pl.BlockSpec((1, D), lambda i: (0, 0)),
        ],
        out_specs=pl.BlockSpec((BN, D), lambda i: (i, 0)),
        out_shape=jax.ShapeDtypeStruct((N, D), jnp.float32),
    )(x, agg2, out_g.reshape(1, D), out_b.reshape(1, D))


def kernel(x, edge_index, edge_attr, W1, b1, ln1_g, ln1_b, W2, b2, ln2_g,
           ln2_b, out_g, out_b):
    N, D = x.shape
    E, ED = edge_attr.shape
    NW, GROUP = 32, 128

    conv16 = _mlp_conv16(x, W1, b1, ln1_g, ln1_b, W2, b2, ln2_g, ln2_b, ED)

    if E % GROUP:
        pad = GROUP - E % GROUP
        edge_index = jnp.pad(edge_index, ((0, 0), (0, pad)))
        edge_attr = jnp.pad(edge_attr, ((0, pad), (0, 0)))
        E += pad
    ei3 = edge_index.reshape(2, E // GROUP, GROUP)

    N_pad = -(-N // (16 * 8)) * (16 * 8)
    agg2 = _sc_edge_agg(conv16, ei3, edge_attr, N_pad)
    return _final_ln(x, agg2, out_g, out_b)
